# baseline (speedup 1.0000x reference)
import functools

import jax
import jax.numpy as jnp
from jax import lax
from jax.experimental import pallas as pl
from jax.experimental.pallas import tpu as pltpu
from jax.experimental.pallas import tpu_sc as plsc

DIM = 32
FIELDS = 26
NF = 13
BBLK = 128
NBLK = 8
NPAIR = 6


def _gather_kernel(t4_hbm, idx_hbm, out_hbm,
                   xi_v, r0, r1, qb0, qb1, rows0, rows1, stage_v,
                   gsem0, gsem1, osem):
    w = lax.axis_index("s") * 2 + lax.axis_index("c")
    h = w // 16
    bg = w % 16
    f0 = h * NF
    rbuf = (r0, r1)
    qbuf = (qb0, qb1)
    rows = (rows0, rows1)
    gsem = (gsem0, gsem1)

    iota = lax.iota(jnp.int32, 16)

    def build_rq(f, pb):
        r_v, qb_v = rbuf[pb], qbuf[pb]

        def g_body(g, carry):
            p16 = (iota + g * 16) * FIELDS + (f0 + f)
            c = plsc.load_gather(xi_v, [p16])
            r_v[pl.ds(g * 16, 16)] = lax.shift_right_logical(c, 2)
            qb_v[pl.ds(g * 16, 16)] = lax.shift_left(jnp.bitwise_and(c, 3), 5)
            return carry
        lax.fori_loop(0, BBLK // 16, g_body, 0)

    def issue(pb):
        return pltpu.async_copy(t4_hbm.at[rbuf[pb]], rows[pb], gsem[pb])

    def wait(pb):
        pltpu.make_async_copy(t4_hbm.at[rbuf[pb]], rows[pb], gsem[pb]).wait()

    def extract(f, bb0, pb):
        rows_b, qb_v = rows[pb], qbuf[pb]
        for g in range(BBLK // 16):
            b16 = iota + g * 16
            qb16 = qb_v[pl.ds(g * 16, 16)]
            for d in range(DIM):
                stage_v[d, pl.ds(g * 16, 16)] = plsc.load_gather(
                    rows_b, [b16, qb16 + d])
        pltpu.sync_copy(stage_v, out_hbm.at[f0 + f, :, pl.ds(bb0, BBLK)])

    def block_body(blk, carry):
        bb0 = bg * (NBLK * BBLK) + blk * BBLK
        pltpu.sync_copy(idx_hbm.at[pl.ds(bb0 * FIELDS, BBLK * FIELDS)], xi_v)

        build_rq(0, 0)
        issue(0)

        def pair_body(i, carry2):
            c = 2 * i
            build_rq(c + 1, 1)
            issue(1)
            wait(0)
            extract(c, bb0, 0)
            build_rq(c + 2, 0)
            issue(0)
            wait(1)
            extract(c + 1, bb0, 1)
            return carry2
        lax.fori_loop(0, NPAIR, pair_body, 0)

        wait(0)
        extract(NF - 1, bb0, 0)
        return carry

    lax.fori_loop(0, NBLK, block_body, 0)


def kernel(x, embedding):
    batch, fields = x.shape
    t4 = embedding.reshape(250000, 128)
    idx = x.reshape(batch * fields)
    mesh = plsc.VectorSubcoreMesh(core_axis_name="c", subcore_axis_name="s")

    run = pl.kernel(
        _gather_kernel,
        out_type=jax.ShapeDtypeStruct((fields, DIM, batch), jnp.float32),
        mesh=mesh,
        scratch_types=[
            pltpu.VMEM((BBLK * FIELDS,), jnp.int32),
            pltpu.VMEM((BBLK,), jnp.int32),
            pltpu.VMEM((BBLK,), jnp.int32),
            pltpu.VMEM((BBLK,), jnp.int32),
            pltpu.VMEM((BBLK,), jnp.int32),
            pltpu.VMEM((BBLK, 128), jnp.float32),
            pltpu.VMEM((BBLK, 128), jnp.float32),
            pltpu.VMEM((DIM, BBLK), jnp.float32),
            pltpu.SemaphoreType.DMA,
            pltpu.SemaphoreType.DMA,
            pltpu.SemaphoreType.DMA,
        ],
        compiler_params=pltpu.CompilerParams(
            use_tc_tiling_on_sc=True, needs_layout_passes=False),
    )
    out_t = run(t4, idx)
    return jnp.transpose(out_t, (2, 0, 1))
